# Initial kernel scaffold; baseline (speedup 1.0000x reference)
import jax
import jax.numpy as jnp
from jax.experimental import pallas as pl


def kernel(nfeat, efeat, edge_index, W, b):
    raise NotImplementedError("write your pallas kernel here")


# speedup vs baseline: 2.6708x; 2.6708x over previous
import functools

import jax
import jax.numpy as jnp
from jax import lax
from jax.experimental import pallas as pl
from jax.experimental.pallas import tpu as pltpu
from jax.experimental.pallas import tpu_sc as plsc

N_NODES = 10000
N_EDGES = 160000
D = 256
H = 128
NT = 16
E_PER_TILE = N_EDGES // NT
CHUNK = 80
CHUNKS_PER_TILE = E_PER_TILE // CHUNK
ZROWS = 80


def _sc_body(nl_hbm, nr_hbm, efeat_hbm, src_hbm, dst_hbm,
             sum0_hbm, sum1_hbm, cnt_hbm,
             acc_sh, cnt_sh, src_v, dst_v, gbuf, ebuf, ones_v, zrow, zflat,
             sem):
    cid = lax.axis_index("c")
    sid = lax.axis_index("s")
    zero16 = jnp.zeros((16,), jnp.float32)
    one16 = jnp.ones((16,), jnp.float32)

    def zr_body(i, _):
        r = i // 8
        k = (i % 8) * 16
        zrow[r, pl.ds(k, 16)] = zero16
        return 0
    lax.fori_loop(0, ZROWS * 8, zr_body, 0)

    def zf_body(i, _):
        zflat[pl.ds(i * 16, 16)] = zero16
        return 0
    lax.fori_loop(0, 125, zf_body, 0)

    for p in range(5):
        ones_v[pl.ds(p * 16, 16)] = one16

    @pl.when(sid < 15)
    def _():
        for p in range(8):
            pltpu.sync_copy(zrow, acc_sh.at[pl.ds(sid * 640 + p * ZROWS,
                                                  ZROWS)])

    @pl.when(sid == 15)
    def _():
        for p in range(5):
            pltpu.sync_copy(zrow, acc_sh.at[pl.ds(9600 + p * ZROWS, ZROWS)])

    @pl.when((cid == 0) & (sid < 5))
    def _():
        pltpu.sync_copy(zflat, cnt_sh.at[pl.ds(sid * 2000, 2000)])

    plsc.subcore_barrier()

    def process(table_hbm, col0, j):
        e0 = sid * E_PER_TILE + j * CHUNK
        pltpu.sync_copy(src_hbm.at[pl.ds(e0, CHUNK)], src_v)
        pltpu.sync_copy(dst_hbm.at[pl.ds(e0, CHUNK)], dst_v)
        pltpu.async_copy(table_hbm.at[src_v], gbuf, sem).wait()
        pltpu.sync_copy(efeat_hbm.at[pl.ds(e0, CHUNK), pl.ds(col0, H)], ebuf)

        def row_body(r, _):
            for k in range(8):
                c = k * 16
                ebuf[r, pl.ds(c, 16)] = jnp.maximum(
                    ebuf[r, pl.ds(c, 16)] + gbuf[r, pl.ds(c, 16)], 0.0)
            return 0
        lax.fori_loop(0, CHUNK, row_body, 0)

        pltpu.sync_copy(ebuf, acc_sh.at[dst_v], add=True)

    @pl.when(cid == 0)
    def _():
        def body(j, _):
            process(nl_hbm, 0, j)
            pltpu.sync_copy(ones_v, cnt_sh.at[dst_v], add=True)
            return 0
        lax.fori_loop(0, CHUNKS_PER_TILE, body, 0)

    @pl.when(cid == 1)
    def _():
        def body(j, _):
            process(nr_hbm, H, j)
            return 0
        lax.fori_loop(0, CHUNKS_PER_TILE, body, 0)

    plsc.subcore_barrier()

    for out_cid, out_hbm in ((0, sum0_hbm), (1, sum1_hbm)):
        @pl.when((cid == out_cid) & (sid < 15))
        def _(out_hbm=out_hbm):
            pltpu.sync_copy(acc_sh.at[pl.ds(sid * 640, 640)],
                            out_hbm.at[pl.ds(sid * 640, 640)])

        @pl.when((cid == out_cid) & (sid == 15))
        def _(out_hbm=out_hbm):
            pltpu.sync_copy(acc_sh.at[pl.ds(9600, 400)],
                            out_hbm.at[pl.ds(9600, 400)])

    @pl.when((cid == 0) & (sid < 5))
    def _():
        pltpu.sync_copy(cnt_sh.at[pl.ds(sid * 2000, 2000)], zflat)
        pltpu.sync_copy(zflat, cnt_hbm.at[pl.ds(sid * 2000, 2000)])


_sc_aggregate = functools.partial(
    pl.kernel,
    out_type=(
        jax.ShapeDtypeStruct((N_NODES, H), jnp.float32),
        jax.ShapeDtypeStruct((N_NODES, H), jnp.float32),
        jax.ShapeDtypeStruct((N_NODES,), jnp.float32),
    ),
    mesh=plsc.VectorSubcoreMesh(core_axis_name="c", subcore_axis_name="s"),
    scratch_types=(
        pltpu.VMEM_SHARED((N_NODES, H), jnp.float32),
        pltpu.VMEM_SHARED((N_NODES,), jnp.float32),
        pltpu.VMEM((CHUNK,), jnp.int32),
        pltpu.VMEM((CHUNK,), jnp.int32),
        pltpu.VMEM((CHUNK, H), jnp.float32),
        pltpu.VMEM((CHUNK, H), jnp.float32),
        pltpu.VMEM((CHUNK,), jnp.float32),
        pltpu.VMEM((ZROWS, H), jnp.float32),
        pltpu.VMEM((2000,), jnp.float32),
        pltpu.SemaphoreType.DMA,
    ),
)(_sc_body)


BLK = 1000


def _tc_body(cnt_ref, s0_ref, s1_ref, nl_ref, nr_ref, wt0_ref, wt1_ref,
             b_ref, out_ref):
    r = 1.0 / jnp.maximum(cnt_ref[...], 1.0)
    x0 = s0_ref[...] * r + nl_ref[...]
    x1 = s1_ref[...] * r + nr_ref[...]
    acc = jnp.dot(x0, wt0_ref[...], preferred_element_type=jnp.float32,
                  precision=lax.Precision.HIGHEST)
    acc = acc + jnp.dot(x1, wt1_ref[...], preferred_element_type=jnp.float32,
                        precision=lax.Precision.HIGHEST)
    out_ref[...] = acc + b_ref[...]


_tc_update = pl.pallas_call(
    _tc_body,
    grid=(N_NODES // BLK,),
    in_specs=[
        pl.BlockSpec((BLK, 1), lambda i: (i, 0)),
        pl.BlockSpec((BLK, H), lambda i: (i, 0)),
        pl.BlockSpec((BLK, H), lambda i: (i, 0)),
        pl.BlockSpec((BLK, H), lambda i: (i, 0)),
        pl.BlockSpec((BLK, H), lambda i: (i, 0)),
        pl.BlockSpec((H, D), lambda i: (0, 0)),
        pl.BlockSpec((H, D), lambda i: (0, 0)),
        pl.BlockSpec((1, D), lambda i: (0, 0)),
    ],
    out_specs=pl.BlockSpec((BLK, D), lambda i: (i, 0)),
    out_shape=jax.ShapeDtypeStruct((N_NODES, D), jnp.float32),
)


def kernel(nfeat, efeat, edge_index, W, b):
    src = edge_index[0].astype(jnp.int32)
    dst = edge_index[1].astype(jnp.int32)
    nl = nfeat[:, :H]
    nr = nfeat[:, H:]
    sum0, sum1, cnt = _sc_aggregate(nl, nr, efeat, src, dst)
    wt = W.T
    return _tc_update(cnt[:, None], sum0, sum1, nl, nr,
                      wt[:H], wt[H:], b[None, :])


# speedup vs baseline: 5.8208x; 2.1794x over previous
import functools

import jax
import jax.numpy as jnp
from jax import lax
from jax.experimental import pallas as pl
from jax.experimental.pallas import tpu as pltpu
from jax.experimental.pallas import tpu_sc as plsc

N_NODES = 10000
N_EDGES = 160000
D = 256
H = 128
NT = 16
E_PER_TILE = N_EDGES // NT
CHUNK = 80
NCH = E_PER_TILE // CHUNK
ZROWS = 80


def _sc_body(nl_hbm, nr_hbm, efeat_hbm, src_hbm, dst_hbm,
             sum0_hbm, sum1_hbm, cnt_hbm,
             acc_sh, cnt_sh, srcring, dstring, gbuf, ebuf, ones_v, zflat,
             sem_in0, sem_in1, sem_sc0, sem_sc1, sem_ix0, sem_ix1):
    cid = lax.axis_index("c")
    sid = lax.axis_index("s")
    zero16 = jnp.zeros((16,), jnp.float32)
    one16 = jnp.ones((16,), jnp.float32)
    sem_in = (sem_in0, sem_in1)
    sem_sc = (sem_sc0, sem_sc1)
    sem_ix = (sem_ix0, sem_ix1)

    @plsc.parallel_loop(0, ZROWS * 8)
    def _(i):
        ebuf[0, i // 8, pl.ds((i % 8) * 16, 16)] = zero16

    @plsc.parallel_loop(0, 125)
    def _(i):
        zflat[pl.ds(i * 16, 16)] = zero16

    for p in range(5):
        ones_v[pl.ds(p * 16, 16)] = one16

    @pl.when(sid < 15)
    def _():
        for p in range(8):
            pltpu.sync_copy(ebuf.at[0], acc_sh.at[pl.ds(sid * 640 + p * ZROWS,
                                                        ZROWS)])

    @pl.when(sid == 15)
    def _():
        for p in range(5):
            pltpu.sync_copy(ebuf.at[0], acc_sh.at[pl.ds(9600 + p * ZROWS,
                                                        ZROWS)])

    @pl.when((cid == 0) & (sid < 5))
    def _():
        pltpu.sync_copy(zflat, cnt_sh.at[pl.ds(sid * 2000, 2000)])

    plsc.subcore_barrier()

    e_base = sid * E_PER_TILE

    def fire_idx(j, p):
        e0 = e_base + j * CHUNK
        r = j % 4
        pltpu.async_copy(src_hbm.at[pl.ds(e0, CHUNK)], srcring.at[r],
                         sem_ix[p])
        pltpu.async_copy(dst_hbm.at[pl.ds(e0, CHUNK)], dstring.at[r],
                         sem_ix[p])

    def wait_idx(j, p):
        e0 = e_base + j * CHUNK
        r = j % 4
        pltpu.make_async_copy(src_hbm.at[pl.ds(e0, CHUNK)], srcring.at[r],
                              sem_ix[p]).wait()
        pltpu.make_async_copy(dst_hbm.at[pl.ds(e0, CHUNK)], dstring.at[r],
                              sem_ix[p]).wait()

    def run_core(table_hbm, col0, with_cnt):
        def fire(j, b):
            r = j % 4
            pltpu.async_copy(table_hbm.at[srcring.at[r]], gbuf.at[b],
                             sem_in[b])
            e0 = e_base + j * CHUNK
            pltpu.async_copy(
                efeat_hbm.at[pl.ds(e0, CHUNK), pl.ds(col0, H)],
                ebuf.at[b], sem_in[b])

        def wait_in(j, b):
            r = j % 4
            pltpu.make_async_copy(table_hbm.at[srcring.at[r]], gbuf.at[b],
                                  sem_in[b]).wait()
            e0 = e_base + j * CHUNK
            pltpu.make_async_copy(
                efeat_hbm.at[pl.ds(e0, CHUNK), pl.ds(col0, H)],
                ebuf.at[b], sem_in[b]).wait()

        def compute(b):
            @plsc.parallel_loop(0, CHUNK, unroll=2)
            def _(r):
                for k in range(8):
                    c = k * 16
                    ebuf[b, r, pl.ds(c, 16)] = jnp.maximum(
                        ebuf[b, r, pl.ds(c, 16)] + gbuf[b, r, pl.ds(c, 16)],
                        0.0)

        def scatter(j, b):
            r = j % 4
            pltpu.async_copy(ebuf.at[b], acc_sh.at[dstring.at[r]],
                             sem_sc[b], add=True)
            if with_cnt:
                pltpu.async_copy(ones_v, cnt_sh.at[dstring.at[r]],
                                 sem_sc[b], add=True)

        def wait_sc(j, b):
            r = j % 4
            pltpu.make_async_copy(ebuf.at[b], acc_sh.at[dstring.at[r]],
                                  sem_sc[b]).wait()
            if with_cnt:
                pltpu.make_async_copy(ones_v, cnt_sh.at[dstring.at[r]],
                                      sem_sc[b]).wait()

        pltpu.sync_copy(src_hbm.at[pl.ds(e_base, CHUNK)], srcring.at[0])
        pltpu.sync_copy(dst_hbm.at[pl.ds(e_base, CHUNK)], dstring.at[0])
        fire(0, 0)
        fire_idx(1, 1)

        def pair(g, _):
            j0 = 2 * g
            j1 = j0 + 1
            wait_in(j0, 0)

            @pl.when(g > 0)
            def _():
                wait_sc(j0 - 1, 1)

            wait_idx(j1, 1)
            fire(j1, 1)
            fire_idx(j0 + 2, 0)
            compute(0)
            scatter(j0, 0)
            wait_in(j1, 1)
            wait_sc(j0, 0)
            wait_idx(j0 + 2, 0)
            fire(j0 + 2, 0)
            compute(1)
            scatter(j1, 1)

            @pl.when(g < (NCH - 1) // 2 - 1)
            def _():
                fire_idx(j1 + 2, 1)
            return 0

        lax.fori_loop(0, (NCH - 1) // 2, pair, 0)
        wait_in(NCH - 1, 0)
        wait_sc(NCH - 2, 1)
        compute(0)
        scatter(NCH - 1, 0)
        wait_sc(NCH - 1, 0)

    @pl.when(cid == 0)
    def _():
        run_core(nl_hbm, 0, True)

    @pl.when(cid == 1)
    def _():
        run_core(nr_hbm, H, False)

    plsc.subcore_barrier()

    for out_cid, out_hbm in ((0, sum0_hbm), (1, sum1_hbm)):
        @pl.when((cid == out_cid) & (sid < 15))
        def _(out_hbm=out_hbm):
            pltpu.sync_copy(acc_sh.at[pl.ds(sid * 640, 640)],
                            out_hbm.at[pl.ds(sid * 640, 640)])

        @pl.when((cid == out_cid) & (sid == 15))
        def _(out_hbm=out_hbm):
            pltpu.sync_copy(acc_sh.at[pl.ds(9600, 400)],
                            out_hbm.at[pl.ds(9600, 400)])

    @pl.when((cid == 0) & (sid < 5))
    def _():
        pltpu.sync_copy(cnt_sh.at[pl.ds(sid * 2000, 2000)], zflat)
        pltpu.sync_copy(zflat, cnt_hbm.at[pl.ds(sid * 2000, 2000)])


_sc_aggregate = functools.partial(
    pl.kernel,
    out_type=(
        jax.ShapeDtypeStruct((N_NODES, H), jnp.float32),
        jax.ShapeDtypeStruct((N_NODES, H), jnp.float32),
        jax.ShapeDtypeStruct((N_NODES,), jnp.float32),
    ),
    mesh=plsc.VectorSubcoreMesh(core_axis_name="c", subcore_axis_name="s"),
    scratch_types=(
        pltpu.VMEM_SHARED((N_NODES, H), jnp.float32),
        pltpu.VMEM_SHARED((N_NODES,), jnp.float32),
        pltpu.VMEM((4, CHUNK), jnp.int32),
        pltpu.VMEM((4, CHUNK), jnp.int32),
        pltpu.VMEM((2, CHUNK, H), jnp.float32),
        pltpu.VMEM((2, CHUNK, H), jnp.float32),
        pltpu.VMEM((CHUNK,), jnp.float32),
        pltpu.VMEM((2000,), jnp.float32),
        pltpu.SemaphoreType.DMA,
        pltpu.SemaphoreType.DMA,
        pltpu.SemaphoreType.DMA,
        pltpu.SemaphoreType.DMA,
        pltpu.SemaphoreType.DMA,
        pltpu.SemaphoreType.DMA,
    ),
)(_sc_body)


BLK = 1000


def _tc_body(cnt_ref, s0_ref, s1_ref, nl_ref, nr_ref, wt0_ref, wt1_ref,
             b_ref, out_ref):
    r = 1.0 / jnp.maximum(cnt_ref[...], 1.0)
    x0 = s0_ref[...] * r + nl_ref[...]
    x1 = s1_ref[...] * r + nr_ref[...]
    acc = jnp.dot(x0, wt0_ref[...], preferred_element_type=jnp.float32,
                  precision=lax.Precision.HIGHEST)
    acc = acc + jnp.dot(x1, wt1_ref[...], preferred_element_type=jnp.float32,
                        precision=lax.Precision.HIGHEST)
    out_ref[...] = acc + b_ref[...]


_tc_update = pl.pallas_call(
    _tc_body,
    grid=(N_NODES // BLK,),
    in_specs=[
        pl.BlockSpec((BLK, 1), lambda i: (i, 0)),
        pl.BlockSpec((BLK, H), lambda i: (i, 0)),
        pl.BlockSpec((BLK, H), lambda i: (i, 0)),
        pl.BlockSpec((BLK, H), lambda i: (i, 0)),
        pl.BlockSpec((BLK, H), lambda i: (i, 0)),
        pl.BlockSpec((H, D), lambda i: (0, 0)),
        pl.BlockSpec((H, D), lambda i: (0, 0)),
        pl.BlockSpec((1, D), lambda i: (0, 0)),
    ],
    out_specs=pl.BlockSpec((BLK, D), lambda i: (i, 0)),
    out_shape=jax.ShapeDtypeStruct((N_NODES, D), jnp.float32),
)


def kernel(nfeat, efeat, edge_index, W, b):
    src = edge_index[0].astype(jnp.int32)
    dst = edge_index[1].astype(jnp.int32)
    nl = nfeat[:, :H]
    nr = nfeat[:, H:]
    sum0, sum1, cnt = _sc_aggregate(nl, nr, efeat, src, dst)
    wt = W.T
    return _tc_update(cnt[:, None], sum0, sum1, nl, nr,
                      wt[:H], wt[H:], b[None, :])


# speedup vs baseline: 6.0081x; 1.0322x over previous
import functools

import jax
import jax.numpy as jnp
from jax import lax
from jax.experimental import pallas as pl
from jax.experimental.pallas import tpu as pltpu
from jax.experimental.pallas import tpu_sc as plsc

N_NODES = 10000
N_EDGES = 160000
D = 256
H = 128
NT = 16
E_PER_TILE = N_EDGES // NT
CHUNK = 80
NCH = E_PER_TILE // CHUNK
ZROWS = 80


def _sc_body(nl_hbm, nr_hbm, efeat_hbm, src_hbm, dst_hbm,
             sum0_hbm, sum1_hbm, cnt_hbm,
             acc_sh, cnt_sh, srcring, dstring, gbuf, ebuf, ones_v, zflat,
             sem_in0, sem_in1, sem_sc0, sem_sc1, sem_ix0, sem_ix1):
    cid = lax.axis_index("c")
    sid = lax.axis_index("s")
    zero16 = jnp.zeros((16,), jnp.float32)
    one16 = jnp.ones((16,), jnp.float32)
    sem_in = (sem_in0, sem_in1)
    sem_sc = (sem_sc0, sem_sc1)
    sem_ix = (sem_ix0, sem_ix1)

    @plsc.parallel_loop(0, ZROWS * 8)
    def _(i):
        ebuf[0, i // 8, pl.ds((i % 8) * 16, 16)] = zero16

    @plsc.parallel_loop(0, 125)
    def _(i):
        zflat[pl.ds(i * 16, 16)] = zero16

    for p in range(5):
        ones_v[pl.ds(p * 16, 16)] = one16

    @pl.when(sid < 15)
    def _():
        for p in range(8):
            pltpu.sync_copy(ebuf.at[0], acc_sh.at[pl.ds(sid * 640 + p * ZROWS,
                                                        ZROWS)])

    @pl.when(sid == 15)
    def _():
        for p in range(5):
            pltpu.sync_copy(ebuf.at[0], acc_sh.at[pl.ds(9600 + p * ZROWS,
                                                        ZROWS)])

    @pl.when((cid == 0) & (sid < 5))
    def _():
        pltpu.sync_copy(zflat, cnt_sh.at[pl.ds(sid * 2000, 2000)])

    plsc.subcore_barrier()

    e_base = sid * E_PER_TILE

    def fire_idx(j, p):
        e0 = e_base + j * CHUNK
        r = j % 4
        pltpu.async_copy(src_hbm.at[pl.ds(e0, CHUNK)], srcring.at[r],
                         sem_ix[p])
        pltpu.async_copy(dst_hbm.at[pl.ds(e0, CHUNK)], dstring.at[r],
                         sem_ix[p])

    def wait_idx(j, p):
        e0 = e_base + j * CHUNK
        r = j % 4
        pltpu.make_async_copy(src_hbm.at[pl.ds(e0, CHUNK)], srcring.at[r],
                              sem_ix[p]).wait()
        pltpu.make_async_copy(dst_hbm.at[pl.ds(e0, CHUNK)], dstring.at[r],
                              sem_ix[p]).wait()

    def run_core(table_hbm, col0, with_cnt):
        def fire(j, b):
            r = j % 4
            pltpu.async_copy(table_hbm.at[srcring.at[r]], gbuf.at[b],
                             sem_in[b])
            e0 = e_base + j * CHUNK
            pltpu.async_copy(
                efeat_hbm.at[pl.ds(e0, CHUNK), pl.ds(col0, H)],
                ebuf.at[b], sem_in[b])

        def wait_in(j, b):
            r = j % 4
            pltpu.make_async_copy(table_hbm.at[srcring.at[r]], gbuf.at[b],
                                  sem_in[b]).wait()
            e0 = e_base + j * CHUNK
            pltpu.make_async_copy(
                efeat_hbm.at[pl.ds(e0, CHUNK), pl.ds(col0, H)],
                ebuf.at[b], sem_in[b]).wait()

        def compute(b):
            @plsc.parallel_loop(0, CHUNK, unroll=2)
            def _(r):
                for k in range(8):
                    c = k * 16
                    ebuf[b, r, pl.ds(c, 16)] = jnp.maximum(
                        ebuf[b, r, pl.ds(c, 16)] + gbuf[b, r, pl.ds(c, 16)],
                        0.0)

        def scatter(j, b):
            r = j % 4
            pltpu.async_copy(ebuf.at[b], acc_sh.at[dstring.at[r]],
                             sem_sc[b], add=True)
            if with_cnt:
                pltpu.async_copy(ones_v, cnt_sh.at[dstring.at[r]],
                                 sem_sc[b], add=True)

        def wait_sc(j, b):
            r = j % 4
            pltpu.make_async_copy(ebuf.at[b], acc_sh.at[dstring.at[r]],
                                  sem_sc[b]).wait()
            if with_cnt:
                pltpu.make_async_copy(ones_v, cnt_sh.at[dstring.at[r]],
                                      sem_sc[b]).wait()

        pltpu.sync_copy(src_hbm.at[pl.ds(e_base, CHUNK)], srcring.at[0])
        pltpu.sync_copy(dst_hbm.at[pl.ds(e_base, CHUNK)], dstring.at[0])
        fire(0, 0)
        fire_idx(1, 1)

        def pair(g, _):
            j0 = 2 * g
            j1 = j0 + 1
            wait_in(j0, 0)

            @pl.when(g > 0)
            def _():
                wait_sc(j0 - 1, 1)

            wait_idx(j1, 1)
            fire(j1, 1)
            fire_idx(j0 + 2, 0)
            compute(0)
            scatter(j0, 0)
            wait_in(j1, 1)
            wait_sc(j0, 0)
            wait_idx(j0 + 2, 0)
            fire(j0 + 2, 0)
            compute(1)
            scatter(j1, 1)

            @pl.when(g < (NCH - 1) // 2 - 1)
            def _():
                fire_idx(j1 + 2, 1)
            return 0

        lax.fori_loop(0, (NCH - 1) // 2, pair, 0)
        wait_in(NCH - 1, 0)
        wait_sc(NCH - 2, 1)
        compute(0)
        scatter(NCH - 1, 0)
        wait_sc(NCH - 1, 0)

    @pl.when(cid == 0)
    def _():
        run_core(nl_hbm, 0, True)

    @pl.when(cid == 1)
    def _():
        run_core(nr_hbm, H, False)

    plsc.subcore_barrier()

    for out_cid, out_hbm in ((0, sum0_hbm), (1, sum1_hbm)):
        @pl.when((cid == out_cid) & (sid < 15))
        def _(out_hbm=out_hbm):
            pltpu.sync_copy(acc_sh.at[pl.ds(sid * 640, 640)],
                            out_hbm.at[pl.ds(sid * 640, 640)])

        @pl.when((cid == out_cid) & (sid == 15))
        def _(out_hbm=out_hbm):
            pltpu.sync_copy(acc_sh.at[pl.ds(9600, 400)],
                            out_hbm.at[pl.ds(9600, 400)])

    @pl.when((cid == 0) & (sid < 5))
    def _():
        pltpu.sync_copy(cnt_sh.at[pl.ds(sid * 2000, 2000)], zflat)
        pltpu.sync_copy(zflat, cnt_hbm.at[pl.ds(sid * 2000, 2000)])


_sc_aggregate = functools.partial(
    pl.kernel,
    out_type=(
        jax.ShapeDtypeStruct((N_NODES, H), jnp.float32),
        jax.ShapeDtypeStruct((N_NODES, H), jnp.float32),
        jax.ShapeDtypeStruct((N_NODES,), jnp.float32),
    ),
    mesh=plsc.VectorSubcoreMesh(core_axis_name="c", subcore_axis_name="s"),
    scratch_types=(
        pltpu.VMEM_SHARED((N_NODES, H), jnp.float32),
        pltpu.VMEM_SHARED((N_NODES,), jnp.float32),
        pltpu.VMEM((4, CHUNK), jnp.int32),
        pltpu.VMEM((4, CHUNK), jnp.int32),
        pltpu.VMEM((2, CHUNK, H), jnp.float32),
        pltpu.VMEM((2, CHUNK, H), jnp.float32),
        pltpu.VMEM((CHUNK,), jnp.float32),
        pltpu.VMEM((2000,), jnp.float32),
        pltpu.SemaphoreType.DMA,
        pltpu.SemaphoreType.DMA,
        pltpu.SemaphoreType.DMA,
        pltpu.SemaphoreType.DMA,
        pltpu.SemaphoreType.DMA,
        pltpu.SemaphoreType.DMA,
    ),
)(_sc_body)


BLK = 2000


def _tc_body(cnt_ref, s0_ref, s1_ref, nl_ref, nr_ref, wt0_ref, wt1_ref,
             b_ref, out_ref):
    r = 1.0 / jnp.maximum(cnt_ref[...], 1.0)
    x0 = s0_ref[...] * r + nl_ref[...]
    x1 = s1_ref[...] * r + nr_ref[...]
    acc = jnp.dot(x0, wt0_ref[...], preferred_element_type=jnp.float32)
    acc = acc + jnp.dot(x1, wt1_ref[...], preferred_element_type=jnp.float32)
    out_ref[...] = acc + b_ref[...]


_tc_update = pl.pallas_call(
    _tc_body,
    grid=(N_NODES // BLK,),
    in_specs=[
        pl.BlockSpec((BLK, 1), lambda i: (i, 0)),
        pl.BlockSpec((BLK, H), lambda i: (i, 0)),
        pl.BlockSpec((BLK, H), lambda i: (i, 0)),
        pl.BlockSpec((BLK, H), lambda i: (i, 0)),
        pl.BlockSpec((BLK, H), lambda i: (i, 0)),
        pl.BlockSpec((H, D), lambda i: (0, 0)),
        pl.BlockSpec((H, D), lambda i: (0, 0)),
        pl.BlockSpec((1, D), lambda i: (0, 0)),
    ],
    out_specs=pl.BlockSpec((BLK, D), lambda i: (i, 0)),
    out_shape=jax.ShapeDtypeStruct((N_NODES, D), jnp.float32),
)


def kernel(nfeat, efeat, edge_index, W, b):
    src = edge_index[0].astype(jnp.int32)
    dst = edge_index[1].astype(jnp.int32)
    nl = nfeat[:, :H]
    nr = nfeat[:, H:]
    sum0, sum1, cnt = _sc_aggregate(nl, nr, efeat, src, dst)
    wt = W.T
    return _tc_update(cnt[:, None], sum0, sum1, nl, nr,
                      wt[:H], wt[H:], b[None, :])


# speedup vs baseline: 6.0326x; 1.0041x over previous
import functools

import jax
import jax.numpy as jnp
from jax import lax
from jax.experimental import pallas as pl
from jax.experimental.pallas import tpu as pltpu
from jax.experimental.pallas import tpu_sc as plsc

N_NODES = 10000
N_EDGES = 160000
D = 256
H = 128
NT = 16
E_PER_TILE = N_EDGES // NT
CHUNK = 80
NCH = E_PER_TILE // CHUNK
ZROWS = 80


def _sc_body(nl_hbm, nr_hbm, efeat_hbm, src_hbm, dst_hbm,
             sum0_hbm, sum1_hbm, cnt_hbm,
             acc_sh, cnt_sh, srcring, dstring, gbuf, ebuf, ones_v, zflat,
             sem_in0, sem_in1, sem_sc0, sem_sc1, sem_ix0, sem_ix1):
    cid = lax.axis_index("c")
    sid = lax.axis_index("s")
    zero16 = jnp.zeros((16,), jnp.float32)
    one16 = jnp.ones((16,), jnp.float32)
    sem_in = (sem_in0, sem_in1)
    sem_sc = (sem_sc0, sem_sc1)
    sem_ix = (sem_ix0, sem_ix1)

    @plsc.parallel_loop(0, ZROWS * 8)
    def _(i):
        ebuf[0, i // 8, pl.ds((i % 8) * 16, 16)] = zero16

    @plsc.parallel_loop(0, 125)
    def _(i):
        zflat[pl.ds(i * 16, 16)] = zero16

    for p in range(5):
        ones_v[pl.ds(p * 16, 16)] = one16

    @pl.when(sid < 15)
    def _():
        for p in range(8):
            pltpu.sync_copy(ebuf.at[0], acc_sh.at[pl.ds(sid * 640 + p * ZROWS,
                                                        ZROWS)])

    @pl.when(sid == 15)
    def _():
        for p in range(5):
            pltpu.sync_copy(ebuf.at[0], acc_sh.at[pl.ds(9600 + p * ZROWS,
                                                        ZROWS)])

    @pl.when((cid == 0) & (sid < 5))
    def _():
        pltpu.sync_copy(zflat, cnt_sh.at[pl.ds(sid * 2000, 2000)])

    plsc.subcore_barrier()

    e_base = sid * E_PER_TILE

    def fire_idx(j, p):
        e0 = e_base + j * CHUNK
        r = j % 4
        pltpu.async_copy(src_hbm.at[pl.ds(e0, CHUNK)], srcring.at[r],
                         sem_ix[p])
        pltpu.async_copy(dst_hbm.at[pl.ds(e0, CHUNK)], dstring.at[r],
                         sem_ix[p])

    def wait_idx(j, p):
        e0 = e_base + j * CHUNK
        r = j % 4
        pltpu.make_async_copy(src_hbm.at[pl.ds(e0, CHUNK)], srcring.at[r],
                              sem_ix[p]).wait()
        pltpu.make_async_copy(dst_hbm.at[pl.ds(e0, CHUNK)], dstring.at[r],
                              sem_ix[p]).wait()

    def run_core(table_hbm, col0, with_cnt):
        def fire(j, b):
            r = j % 4
            pltpu.async_copy(table_hbm.at[srcring.at[r]], gbuf.at[b],
                             sem_in[b])
            e0 = e_base + j * CHUNK
            pltpu.async_copy(
                efeat_hbm.at[pl.ds(e0, CHUNK), pl.ds(col0, H)],
                ebuf.at[b], sem_in[b])

        def wait_in(j, b):
            r = j % 4
            pltpu.make_async_copy(table_hbm.at[srcring.at[r]], gbuf.at[b],
                                  sem_in[b]).wait()
            e0 = e_base + j * CHUNK
            pltpu.make_async_copy(
                efeat_hbm.at[pl.ds(e0, CHUNK), pl.ds(col0, H)],
                ebuf.at[b], sem_in[b]).wait()

        def compute(b):
            @plsc.parallel_loop(0, CHUNK, unroll=4)
            def _(r):
                for k in range(8):
                    c = k * 16
                    ebuf[b, r, pl.ds(c, 16)] = jnp.maximum(
                        ebuf[b, r, pl.ds(c, 16)] + gbuf[b, r, pl.ds(c, 16)],
                        0.0)

        def scatter(j, b):
            r = j % 4
            pltpu.async_copy(ebuf.at[b], acc_sh.at[dstring.at[r]],
                             sem_sc[b], add=True)
            if with_cnt:
                pltpu.async_copy(ones_v, cnt_sh.at[dstring.at[r]],
                                 sem_sc[b], add=True)

        def wait_sc(j, b):
            r = j % 4
            pltpu.make_async_copy(ebuf.at[b], acc_sh.at[dstring.at[r]],
                                  sem_sc[b]).wait()
            if with_cnt:
                pltpu.make_async_copy(ones_v, cnt_sh.at[dstring.at[r]],
                                      sem_sc[b]).wait()

        pltpu.sync_copy(src_hbm.at[pl.ds(e_base, CHUNK)], srcring.at[0])
        pltpu.sync_copy(dst_hbm.at[pl.ds(e_base, CHUNK)], dstring.at[0])
        fire(0, 0)
        fire_idx(1, 1)

        def pair(g, _):
            j0 = 2 * g
            j1 = j0 + 1
            wait_in(j0, 0)

            @pl.when(g > 0)
            def _():
                wait_sc(j0 - 1, 1)

            wait_idx(j1, 1)
            fire(j1, 1)
            fire_idx(j0 + 2, 0)
            compute(0)
            scatter(j0, 0)
            wait_in(j1, 1)
            wait_sc(j0, 0)
            wait_idx(j0 + 2, 0)
            fire(j0 + 2, 0)
            compute(1)
            scatter(j1, 1)

            @pl.when(g < (NCH - 1) // 2 - 1)
            def _():
                fire_idx(j1 + 2, 1)
            return 0

        lax.fori_loop(0, (NCH - 1) // 2, pair, 0)
        wait_in(NCH - 1, 0)
        wait_sc(NCH - 2, 1)
        compute(0)
        scatter(NCH - 1, 0)
        wait_sc(NCH - 1, 0)

    @pl.when(cid == 0)
    def _():
        run_core(nl_hbm, 0, True)

    @pl.when(cid == 1)
    def _():
        run_core(nr_hbm, H, False)

    plsc.subcore_barrier()

    for out_cid, out_hbm in ((0, sum0_hbm), (1, sum1_hbm)):
        @pl.when((cid == out_cid) & (sid < 15))
        def _(out_hbm=out_hbm):
            pltpu.sync_copy(acc_sh.at[pl.ds(sid * 640, 640)],
                            out_hbm.at[pl.ds(sid * 640, 640)])

        @pl.when((cid == out_cid) & (sid == 15))
        def _(out_hbm=out_hbm):
            pltpu.sync_copy(acc_sh.at[pl.ds(9600, 400)],
                            out_hbm.at[pl.ds(9600, 400)])

    @pl.when((cid == 0) & (sid < 5))
    def _():
        pltpu.sync_copy(cnt_sh.at[pl.ds(sid * 2000, 2000)], zflat)
        pltpu.sync_copy(zflat, cnt_hbm.at[pl.ds(sid * 2000, 2000)])


_sc_aggregate = functools.partial(
    pl.kernel,
    out_type=(
        jax.ShapeDtypeStruct((N_NODES, H), jnp.float32),
        jax.ShapeDtypeStruct((N_NODES, H), jnp.float32),
        jax.ShapeDtypeStruct((N_NODES,), jnp.float32),
    ),
    mesh=plsc.VectorSubcoreMesh(core_axis_name="c", subcore_axis_name="s"),
    scratch_types=(
        pltpu.VMEM_SHARED((N_NODES, H), jnp.float32),
        pltpu.VMEM_SHARED((N_NODES,), jnp.float32),
        pltpu.VMEM((4, CHUNK), jnp.int32),
        pltpu.VMEM((4, CHUNK), jnp.int32),
        pltpu.VMEM((2, CHUNK, H), jnp.float32),
        pltpu.VMEM((2, CHUNK, H), jnp.float32),
        pltpu.VMEM((CHUNK,), jnp.float32),
        pltpu.VMEM((2000,), jnp.float32),
        pltpu.SemaphoreType.DMA,
        pltpu.SemaphoreType.DMA,
        pltpu.SemaphoreType.DMA,
        pltpu.SemaphoreType.DMA,
        pltpu.SemaphoreType.DMA,
        pltpu.SemaphoreType.DMA,
    ),
)(_sc_body)


BLK = 2000


def _tc_body(cnt_ref, s0_ref, s1_ref, nl_ref, nr_ref, wt0_ref, wt1_ref,
             b_ref, out_ref):
    r = 1.0 / jnp.maximum(cnt_ref[...], 1.0)
    x0 = s0_ref[...] * r + nl_ref[...]
    x1 = s1_ref[...] * r + nr_ref[...]
    acc = jnp.dot(x0, wt0_ref[...], preferred_element_type=jnp.float32)
    acc = acc + jnp.dot(x1, wt1_ref[...], preferred_element_type=jnp.float32)
    out_ref[...] = acc + b_ref[...]


_tc_update = pl.pallas_call(
    _tc_body,
    grid=(N_NODES // BLK,),
    in_specs=[
        pl.BlockSpec((BLK, 1), lambda i: (i, 0)),
        pl.BlockSpec((BLK, H), lambda i: (i, 0)),
        pl.BlockSpec((BLK, H), lambda i: (i, 0)),
        pl.BlockSpec((BLK, H), lambda i: (i, 0)),
        pl.BlockSpec((BLK, H), lambda i: (i, 0)),
        pl.BlockSpec((H, D), lambda i: (0, 0)),
        pl.BlockSpec((H, D), lambda i: (0, 0)),
        pl.BlockSpec((1, D), lambda i: (0, 0)),
    ],
    out_specs=pl.BlockSpec((BLK, D), lambda i: (i, 0)),
    out_shape=jax.ShapeDtypeStruct((N_NODES, D), jnp.float32),
)


def kernel(nfeat, efeat, edge_index, W, b):
    src = edge_index[0].astype(jnp.int32)
    dst = edge_index[1].astype(jnp.int32)
    nl = nfeat[:, :H]
    nr = nfeat[:, H:]
    sum0, sum1, cnt = _sc_aggregate(nl, nr, efeat, src, dst)
    wt = W.T
    return _tc_update(cnt[:, None], sum0, sum1, nl, nr,
                      wt[:H], wt[H:], b[None, :])
